# baseline (speedup 1.0000x reference)
import jax
import jax.numpy as jnp
from jax import lax
from jax.experimental import pallas as pl
from jax.experimental.pallas import tpu as pltpu
from jax.experimental.pallas import tpu_sc as plsc

N = 10000
E = 320000
D = 128
A = 100
BC = 20
G = 256
H = 128
EPS = 1e-3

NC = 2
NS = 16
NW = NC * NS

NP = 10240
ZR = NP // NS
EH = E // 2
GC = 1000
GCS = 200
BE = 4000
BN = 1000


def _gather_body2(table, src_idx, dst_idx, src_out, dst_out, idxs_v, idxt_v,
                  r0, r1, s0, s1, w0, w1):
    wid = lax.axis_index("s") * NC + lax.axis_index("c")
    epw = EH // NW
    base = wid * epw

    def step(j, carry):
        off = base + j * GC
        pltpu.sync_copy(src_idx.at[pl.ds(off, GC)], idxs_v)
        pltpu.sync_copy(dst_idx.at[pl.ds(off, GC)], idxt_v)
        cs = pltpu.async_copy(table.at[idxs_v], r0, s0)
        ct = pltpu.async_copy(table.at[idxt_v], r1, s1)
        cs.wait()
        ws = pltpu.async_copy(r0, src_out.at[pl.ds(off, GC)], w0)
        ct.wait()
        wt = pltpu.async_copy(r1, dst_out.at[pl.ds(off, GC)], w1)
        ws.wait()
        wt.wait()
        return carry

    lax.fori_loop(0, epw // GC, step, 0)


def _sc_gather(table, src_idx, dst_idx):
    mesh = plsc.VectorSubcoreMesh(core_axis_name="c", subcore_axis_name="s")
    epw = EH // NW
    out = jax.ShapeDtypeStruct((EH, D // 2), jnp.int32)
    idx_t = pltpu.VMEM((GC,), jnp.int32)
    buf_t = pltpu.VMEM((GC, D // 2), jnp.int32)
    sem = pltpu.SemaphoreType.DMA
    return pl.kernel(
        _gather_body2,
        out_type=(out, out),
        mesh=mesh,
        scratch_types=[idx_t] * 2 + [buf_t] * 2 + [sem] * 4,
        compiler_params=pltpu.CompilerParams(use_tc_tiling_on_sc=False),
    )(table, src_idx, dst_idx)


def _scatter_body(msg, dst_idx, init0, init1, p0, p1, acc, i0, r0, m0, m1):
    c = lax.axis_index("c")
    s = lax.axis_index("s")

    @pl.when(c == 0)
    def _():
        pltpu.sync_copy(init0.at[pl.ds(s * ZR, ZR)], acc.at[pl.ds(s * ZR, ZR)])

    @pl.when(c == 1)
    def _():
        pltpu.sync_copy(init1.at[pl.ds(s * ZR, ZR)], acc.at[pl.ds(s * ZR, ZR)])

    plsc.subcore_barrier()

    half = EH // NC
    epw = half // NS
    base = c * half + s * epw

    def step(j, carry):
        off = base + j * GCS
        ci = pltpu.async_copy(dst_idx.at[pl.ds(off, GCS)], i0, m0)
        cr = pltpu.async_copy(msg.at[pl.ds(off, GCS)], r0, m1)
        ci.wait()
        cr.wait()
        pltpu.sync_copy(r0, acc.at[i0], add=True)
        return carry

    lax.fori_loop(0, epw // GCS, step, 0)
    plsc.subcore_barrier()

    @pl.when(c == 0)
    def _():
        pltpu.sync_copy(acc.at[pl.ds(s * ZR, ZR)], p0.at[pl.ds(s * ZR, ZR)])

    @pl.when(c == 1)
    def _():
        pltpu.sync_copy(acc.at[pl.ds(s * ZR, ZR)], p1.at[pl.ds(s * ZR, ZR)])


def _sc_scatter(msg, dst_idx, init0, init1):
    mesh = plsc.VectorSubcoreMesh(core_axis_name="c", subcore_axis_name="s")
    return pl.kernel(
        _scatter_body,
        out_type=(
            jax.ShapeDtypeStruct((NP, D), jnp.float32),
            jax.ShapeDtypeStruct((NP, D), jnp.float32),
        ),
        mesh=mesh,
        scratch_types=[
            pltpu.VMEM_SHARED((NP, D), jnp.float32),
            pltpu.VMEM((GCS,), jnp.int32),
            pltpu.VMEM((GCS, D), jnp.float32),
            pltpu.SemaphoreType.DMA,
            pltpu.SemaphoreType.DMA,
        ],
    )(msg, dst_idx, init0, init1)


def _pack_bf16_pair(v):
    a = v[:, :D // 2].astype(jnp.bfloat16).astype(jnp.float32)
    b = v[:, D // 2:].astype(jnp.bfloat16).astype(jnp.float32)
    ai = lax.shift_right_logical(lax.bitcast_convert_type(a, jnp.int32), 16)
    bi = lax.bitcast_convert_type(b, jnp.int32) & jnp.int32(-65536)
    return ai | bi


def _unpack_bf16_pair(x):
    a = lax.bitcast_convert_type(lax.shift_left(x, 16), jnp.float32)
    b = lax.bitcast_convert_type(x & jnp.int32(-65536), jnp.float32)
    return a.astype(jnp.bfloat16), b.astype(jnp.bfloat16)


def _init_kernel_body(types_ref, emb_ref, out_ref, out16_ref):
    t = types_ref[...]
    oh = (t == lax.broadcasted_iota(jnp.int32, (BN, A), 1)).astype(jnp.float32)
    v = jnp.dot(oh, emb_ref[...], preferred_element_type=jnp.float32)
    out_ref[...] = v
    out16_ref[...] = _pack_bf16_pair(v)


def _tc_init(atom_types_2d, atom_emb):
    nb = pl.BlockSpec((BN, D), lambda i: (i, 0))
    return pl.pallas_call(
        _init_kernel_body,
        grid=(N // BN,),
        in_specs=[
            pl.BlockSpec((BN, 1), lambda i: (i, 0)),
            pl.BlockSpec((A, D), lambda i: (0, 0)),
        ],
        out_specs=(nb, pl.BlockSpec((BN, D // 2), lambda i: (i, 0))),
        out_shape=(jax.ShapeDtypeStruct((N, D), jnp.float32),
                   jax.ShapeDtypeStruct((N, D // 2), jnp.int32)),
    )(atom_types_2d, atom_emb)


def _split_pairs(x2):
    return jnp.concatenate([x2[:, :D // 2], x2[:, D // 2:]], axis=0)


def _edge_compute(src2, tgt2, bond16, bond_f32, W1cat, W2, Wa, hb, ab, b2,
                  msg_ref, bond_ref):
    f32 = jnp.float32
    sA, sB = _unpack_bf16_pair(_split_pairs(src2))
    tA, tB = _unpack_bf16_pair(_split_pairs(tgt2))
    sfull = jnp.concatenate([sA, sB], axis=1)
    x = jnp.concatenate([sfull, tA, tB, bond16], axis=1)
    hid = jnp.tanh(jnp.dot(x, W1cat, preferred_element_type=f32) + hb)
    nb = jnp.dot(hid.astype(jnp.bfloat16), W2, preferred_element_type=f32) + b2
    u = jnp.tanh(jnp.dot(sfull, Wa, preferred_element_type=f32) + ab)
    msg_ref[...] = u * nb
    bond_ref[...] = (bond_f32 + nb).astype(jnp.bfloat16)


def _edge_body_first(src_ref, tgt_ref, btyp_ref, bemb_ref, W1cat_ref,
                     W2_ref, Wa_ref, hb_ref, ab_ref, b2_ref,
                     msg_ref, bond_ref):
    bt = btyp_ref[...].reshape(1, BE)
    ohT = (jnp.broadcast_to(bt, (BC, BE))
           == lax.broadcasted_iota(jnp.int32, (BC, BE), 0)).astype(jnp.float32)
    bond = lax.dot_general(ohT, bemb_ref[...], (((0,), (0,)), ((), ())),
                           preferred_element_type=jnp.float32)
    _edge_compute(src_ref[...], tgt_ref[...], bond.astype(jnp.bfloat16),
                  bond, W1cat_ref[...],
                  W2_ref[...], Wa_ref[...], hb_ref[...], ab_ref[...],
                  b2_ref[...], msg_ref, bond_ref)


def _edge_body_next(src_ref, tgt_ref, bin_ref, W1cat_ref,
                    W2_ref, Wa_ref, hb_ref, ab_ref, b2_ref,
                    msg_ref, bond_ref):
    bond16 = bin_ref[...]
    _edge_compute(src_ref[...], tgt_ref[...], bond16,
                  bond16.astype(jnp.float32), W1cat_ref[...],
                  W2_ref[...], Wa_ref[...], hb_ref[...], ab_ref[...],
                  b2_ref[...], msg_ref, bond_ref)


def _full_spec(shape):
    return pl.BlockSpec(shape, lambda i: tuple(0 for _ in shape))


def _tc_edge(first, src2, tgt2, bond_in, weights):
    body = _edge_body_first if first else _edge_body_next
    eb = pl.BlockSpec((BE, D), lambda i: (i, 0))
    ebp = pl.BlockSpec((BE // 2, D), lambda i: (i, 0))
    if first:
        data_specs = [ebp, ebp, pl.BlockSpec((1, 1, BE), lambda i: (i, 0, 0)),
                      _full_spec((BC, D))]
    else:
        data_specs = [ebp, ebp, eb]
    w_specs = [_full_spec((3 * D, 2 * D)), _full_spec((2 * D, D)),
               _full_spec((D, D)),
               _full_spec((1, 2 * D)), _full_spec((1, D)), _full_spec((1, D))]
    return pl.pallas_call(
        body,
        grid=(EH // BE,),
        in_specs=data_specs + w_specs,
        out_specs=(eb, eb),
        out_shape=(
            jax.ShapeDtypeStruct((EH, D), jnp.float32),
            jax.ShapeDtypeStruct((EH, D), jnp.bfloat16),
        ),
    )(src2, tgt2, *bond_in, *weights)


def _combine_body(a_ref, p0_ref, p1_ref, out_ref, out16_ref):
    v = a_ref[...] + p0_ref[...] + p1_ref[...]
    out_ref[...] = v
    out16_ref[...] = _pack_bf16_pair(v)


def _tc_combine(atom, p0, p1):
    nb = pl.BlockSpec((BN, D), lambda i: (i, 0))
    return pl.pallas_call(
        _combine_body,
        grid=(N // BN,),
        in_specs=[nb, nb, nb],
        out_specs=(nb, pl.BlockSpec((BN, D // 2), lambda i: (i, 0))),
        out_shape=(jax.ShapeDtypeStruct((N, D), jnp.float32),
                   jax.ShapeDtypeStruct((N, D // 2), jnp.int32)),
    )(atom, p0, p1)


def _final_body(a_ref, p0_ref, p1_ref, typ_ref, ng_ref, outW_ref, outb_ref,
                finW_ref, finb_ref, mtab_ref, out_ref):
    x = a_ref[...] + p0_ref[...] + p1_ref[...]
    h = jnp.maximum(
        jnp.dot(x, outW_ref[...], preferred_element_type=jnp.float32)
        + outb_ref[...], 0.0)
    e = jnp.dot(h, finW_ref[...], preferred_element_type=jnp.float32) + finb_ref[...]
    t = typ_ref[...]
    ohm = (t == lax.broadcasted_iota(jnp.int32, (BN, A), 1)).astype(jnp.float32)
    e = e + jnp.dot(ohm, mtab_ref[...], preferred_element_type=jnp.float32)
    g_row = ng_ref[...].reshape(1, BN)
    ohg = (jnp.broadcast_to(g_row, (G, BN))
           == lax.broadcasted_iota(jnp.int32, (G, BN), 0)).astype(jnp.float32)
    partial = jnp.dot(ohg, e, preferred_element_type=jnp.float32)

    @pl.when(pl.program_id(0) == 0)
    def _():
        out_ref[...] = jnp.zeros_like(out_ref)

    out_ref[...] += partial


def _tc_final(atom, p0, p1, atom_types_2d, ng_row, out_W, out_b2, final_W,
              final_b2, mtab):
    nb = pl.BlockSpec((BN, D), lambda i: (i, 0))
    return pl.pallas_call(
        _final_body,
        grid=(N // BN,),
        in_specs=[nb, nb, nb,
                  pl.BlockSpec((BN, 1), lambda i: (i, 0)),
                  pl.BlockSpec((1, 1, BN), lambda i: (i, 0, 0)),
                  _full_spec((D, H)), _full_spec((1, H)), _full_spec((H, 1)),
                  _full_spec((1, 1)), _full_spec((A, 1))],
        out_specs=pl.BlockSpec((G, 1), lambda i: (0, 0)),
        out_shape=jax.ShapeDtypeStruct((G, 1), jnp.float32),
    )(atom, p0, p1, atom_types_2d, ng_row, out_W, out_b2, final_W, final_b2,
      mtab)


def kernel(atom_emb, atom_mean_tab, bond_emb, msg_params, out_W, out_b,
           final_W, final_b, atom_types, bond_types, node_graph_indices,
           connectivity):
    inv = 1.0 / jnp.sqrt(1.0 + EPS)
    dst_idx = connectivity[:, 0].astype(jnp.int32)
    src_idx = connectivity[:, 1].astype(jnp.int32)
    bnd_idx = bond_types.astype(jnp.int32)
    atom_types_2d = atom_types.astype(jnp.int32).reshape(N, 1)
    ng_row = node_graph_indices.astype(jnp.int32).reshape(N // BN, 1, BN)
    kp = jnp.concatenate([jnp.arange(0, BE, 2), jnp.arange(1, BE, 2)])
    k_perm = (jnp.arange(E // BE)[:, None] * BE + kp[None, :]).reshape(E)
    dst_idx_k = dst_idx[k_perm]

    step_weights = []
    for p in msg_params:
        sa = p['atom_bn_gamma'] * inv
        ba = p['atom_bn_beta']
        sb = p['bond_bn_gamma'] * inv
        bb = p['bond_bn_beta']
        W1 = p['W1']
        W1s = sa[:, None] * W1[:D]
        W1t = sa[:, None] * W1[D:2 * D]
        W1b = sb[:, None] * W1[2 * D:]
        hb = (ba @ (W1[:D] + W1[D:2 * D]) + bb @ W1[2 * D:]).reshape(1, 2 * D)
        Wa = sa[:, None] * p['Wa']
        ab = (ba @ p['Wa']).reshape(1, D)
        b2 = p['b2'].reshape(1, D)
        bf = jnp.bfloat16
        W1cat = jnp.concatenate([W1s, W1t, W1b], axis=0).astype(bf)
        step_weights.append((W1cat, p['W2'].astype(bf), Wa.astype(bf),
                             hb, ab, b2))

    atom, atom16 = _tc_init(atom_types_2d, atom_emb)
    bond = None
    p0 = p1 = None
    btyp_k = bnd_idx[k_perm].reshape(E // BE, 1, BE)
    znp = jnp.zeros((NP, D), jnp.float32)
    srcH = (src_idx[:EH], src_idx[EH:])
    dstH = (dst_idx[:EH], dst_idx[EH:])
    dstkH = (dst_idx_k[:EH], dst_idx_k[EH:])
    btypH = (btyp_k[:EH // BE], btyp_k[EH // BE:])
    bondH = [None, None]
    for t in range(len(msg_params)):
        inits = (znp, znp)
        for h in range(2):
            src_rows, dst_rows = _sc_gather(atom16, srcH[h], dstH[h])
            bond_in = (btypH[h], bond_emb) if t == 0 else (bondH[h],)
            msg, bondH[h] = _tc_edge(t == 0, src_rows.reshape(EH // 2, D),
                                     dst_rows.reshape(EH // 2, D), bond_in,
                                     step_weights[t])
            inits = _sc_scatter(msg, dstkH[h], *inits)
        p0, p1 = inits
        if t < len(msg_params) - 1:
            atom, atom16 = _tc_combine(atom, p0, p1)

    return _tc_final(atom, p0, p1, atom_types_2d, ng_row, out_W,
                     out_b.reshape(1, H), final_W, final_b.reshape(1, 1),
                     atom_mean_tab)


# speedup vs baseline: 1.0433x; 1.0433x over previous
import jax
import jax.numpy as jnp
from jax import lax
from jax.experimental import pallas as pl
from jax.experimental.pallas import tpu as pltpu
from jax.experimental.pallas import tpu_sc as plsc

N = 10000
E = 320000
D = 128
A = 100
BC = 20
G = 256
H = 128
EPS = 1e-3

NC = 2
NS = 16
NW = NC * NS

NP = 10240
ZR = NP // NS
EH = E // 2
GC = 1000
GCS = 200
BE = 8000
BN = 1000


def _gather_body2(table, src_idx, dst_idx, src_out, dst_out, idxs_v, idxt_v,
                  r0, r1, s0, s1, w0, w1):
    wid = lax.axis_index("s") * NC + lax.axis_index("c")
    epw = EH // NW
    base = wid * epw

    def step(j, carry):
        off = base + j * GC
        pltpu.sync_copy(src_idx.at[pl.ds(off, GC)], idxs_v)
        pltpu.sync_copy(dst_idx.at[pl.ds(off, GC)], idxt_v)
        cs = pltpu.async_copy(table.at[idxs_v], r0, s0)
        ct = pltpu.async_copy(table.at[idxt_v], r1, s1)
        cs.wait()
        ws = pltpu.async_copy(r0, src_out.at[pl.ds(off, GC)], w0)
        ct.wait()
        wt = pltpu.async_copy(r1, dst_out.at[pl.ds(off, GC)], w1)
        ws.wait()
        wt.wait()
        return carry

    lax.fori_loop(0, epw // GC, step, 0)


def _sc_gather(table, src_idx, dst_idx):
    mesh = plsc.VectorSubcoreMesh(core_axis_name="c", subcore_axis_name="s")
    epw = EH // NW
    out = jax.ShapeDtypeStruct((EH, D // 2), jnp.int32)
    idx_t = pltpu.VMEM((GC,), jnp.int32)
    buf_t = pltpu.VMEM((GC, D // 2), jnp.int32)
    sem = pltpu.SemaphoreType.DMA
    return pl.kernel(
        _gather_body2,
        out_type=(out, out),
        mesh=mesh,
        scratch_types=[idx_t] * 2 + [buf_t] * 2 + [sem] * 4,
        compiler_params=pltpu.CompilerParams(use_tc_tiling_on_sc=False),
    )(table, src_idx, dst_idx)


def _scatter_body(msg, dst_idx, init0, init1, p0, p1, acc, i0, r0, m0, m1):
    c = lax.axis_index("c")
    s = lax.axis_index("s")

    @pl.when(c == 0)
    def _():
        pltpu.sync_copy(init0.at[pl.ds(s * ZR, ZR)], acc.at[pl.ds(s * ZR, ZR)])

    @pl.when(c == 1)
    def _():
        pltpu.sync_copy(init1.at[pl.ds(s * ZR, ZR)], acc.at[pl.ds(s * ZR, ZR)])

    plsc.subcore_barrier()

    half = EH // NC
    epw = half // NS
    base = c * half + s * epw

    def step(j, carry):
        off = base + j * GCS
        ci = pltpu.async_copy(dst_idx.at[pl.ds(off, GCS)], i0, m0)
        cr = pltpu.async_copy(msg.at[pl.ds(off, GCS)], r0, m1)
        ci.wait()
        cr.wait()
        pltpu.sync_copy(r0, acc.at[i0], add=True)
        return carry

    lax.fori_loop(0, epw // GCS, step, 0)
    plsc.subcore_barrier()

    @pl.when(c == 0)
    def _():
        pltpu.sync_copy(acc.at[pl.ds(s * ZR, ZR)], p0.at[pl.ds(s * ZR, ZR)])

    @pl.when(c == 1)
    def _():
        pltpu.sync_copy(acc.at[pl.ds(s * ZR, ZR)], p1.at[pl.ds(s * ZR, ZR)])


def _sc_scatter(msg, dst_idx, init0, init1):
    mesh = plsc.VectorSubcoreMesh(core_axis_name="c", subcore_axis_name="s")
    return pl.kernel(
        _scatter_body,
        out_type=(
            jax.ShapeDtypeStruct((NP, D), jnp.float32),
            jax.ShapeDtypeStruct((NP, D), jnp.float32),
        ),
        mesh=mesh,
        scratch_types=[
            pltpu.VMEM_SHARED((NP, D), jnp.float32),
            pltpu.VMEM((GCS,), jnp.int32),
            pltpu.VMEM((GCS, D), jnp.float32),
            pltpu.SemaphoreType.DMA,
            pltpu.SemaphoreType.DMA,
        ],
    )(msg, dst_idx, init0, init1)


def _pack_bf16_pair(v):
    a = v[:, :D // 2].astype(jnp.bfloat16).astype(jnp.float32)
    b = v[:, D // 2:].astype(jnp.bfloat16).astype(jnp.float32)
    ai = lax.shift_right_logical(lax.bitcast_convert_type(a, jnp.int32), 16)
    bi = lax.bitcast_convert_type(b, jnp.int32) & jnp.int32(-65536)
    return ai | bi


def _unpack_bf16_pair(x):
    a = lax.bitcast_convert_type(lax.shift_left(x, 16), jnp.float32)
    b = lax.bitcast_convert_type(x & jnp.int32(-65536), jnp.float32)
    return a.astype(jnp.bfloat16), b.astype(jnp.bfloat16)


def _init_kernel_body(types_ref, emb_ref, out_ref, out16_ref):
    t = types_ref[...]
    oh = (t == lax.broadcasted_iota(jnp.int32, (BN, A), 1)).astype(jnp.float32)
    v = jnp.dot(oh, emb_ref[...], preferred_element_type=jnp.float32)
    out_ref[...] = v
    out16_ref[...] = _pack_bf16_pair(v)


def _tc_init(atom_types_2d, atom_emb):
    nb = pl.BlockSpec((BN, D), lambda i: (i, 0))
    return pl.pallas_call(
        _init_kernel_body,
        grid=(N // BN,),
        in_specs=[
            pl.BlockSpec((BN, 1), lambda i: (i, 0)),
            pl.BlockSpec((A, D), lambda i: (0, 0)),
        ],
        out_specs=(nb, pl.BlockSpec((BN, D // 2), lambda i: (i, 0))),
        out_shape=(jax.ShapeDtypeStruct((N, D), jnp.float32),
                   jax.ShapeDtypeStruct((N, D // 2), jnp.int32)),
    )(atom_types_2d, atom_emb)


def _split_pairs(x2):
    return jnp.concatenate([x2[:, :D // 2], x2[:, D // 2:]], axis=0)


def _edge_compute(src2, tgt2, bond16, bond_f32, W1cat, W2, Wa, hb, ab, b2,
                  msg_ref, bond_ref):
    f32 = jnp.float32
    sA, sB = _unpack_bf16_pair(_split_pairs(src2))
    tA, tB = _unpack_bf16_pair(_split_pairs(tgt2))
    sfull = jnp.concatenate([sA, sB], axis=1)
    x = jnp.concatenate([sfull, tA, tB, bond16], axis=1)
    hid = jnp.tanh(jnp.dot(x, W1cat, preferred_element_type=f32) + hb)
    nb = jnp.dot(hid.astype(jnp.bfloat16), W2, preferred_element_type=f32) + b2
    u = jnp.tanh(jnp.dot(sfull, Wa, preferred_element_type=f32) + ab)
    msg_ref[...] = u * nb
    bond_ref[...] = (bond_f32 + nb).astype(jnp.bfloat16)


def _edge_body_first(src_ref, tgt_ref, btyp_ref, bemb_ref, W1cat_ref,
                     W2_ref, Wa_ref, hb_ref, ab_ref, b2_ref,
                     msg_ref, bond_ref):
    bt = btyp_ref[...].reshape(1, BE)
    ohT = (jnp.broadcast_to(bt, (BC, BE))
           == lax.broadcasted_iota(jnp.int32, (BC, BE), 0)).astype(jnp.float32)
    bond = lax.dot_general(ohT, bemb_ref[...], (((0,), (0,)), ((), ())),
                           preferred_element_type=jnp.float32)
    _edge_compute(src_ref[...], tgt_ref[...], bond.astype(jnp.bfloat16),
                  bond, W1cat_ref[...],
                  W2_ref[...], Wa_ref[...], hb_ref[...], ab_ref[...],
                  b2_ref[...], msg_ref, bond_ref)


def _edge_body_next(src_ref, tgt_ref, bin_ref, W1cat_ref,
                    W2_ref, Wa_ref, hb_ref, ab_ref, b2_ref,
                    msg_ref, bond_ref):
    bond16 = bin_ref[...]
    _edge_compute(src_ref[...], tgt_ref[...], bond16,
                  bond16.astype(jnp.float32), W1cat_ref[...],
                  W2_ref[...], Wa_ref[...], hb_ref[...], ab_ref[...],
                  b2_ref[...], msg_ref, bond_ref)


def _full_spec(shape):
    return pl.BlockSpec(shape, lambda i: tuple(0 for _ in shape))


def _tc_edge(first, src2, tgt2, bond_in, weights):
    body = _edge_body_first if first else _edge_body_next
    eb = pl.BlockSpec((BE, D), lambda i: (i, 0))
    ebp = pl.BlockSpec((BE // 2, D), lambda i: (i, 0))
    if first:
        data_specs = [ebp, ebp, pl.BlockSpec((1, 1, BE), lambda i: (i, 0, 0)),
                      _full_spec((BC, D))]
    else:
        data_specs = [ebp, ebp, eb]
    w_specs = [_full_spec((3 * D, 2 * D)), _full_spec((2 * D, D)),
               _full_spec((D, D)),
               _full_spec((1, 2 * D)), _full_spec((1, D)), _full_spec((1, D))]
    return pl.pallas_call(
        body,
        grid=(EH // BE,),
        in_specs=data_specs + w_specs,
        out_specs=(eb, eb),
        out_shape=(
            jax.ShapeDtypeStruct((EH, D), jnp.float32),
            jax.ShapeDtypeStruct((EH, D), jnp.bfloat16),
        ),
    )(src2, tgt2, *bond_in, *weights)


def _combine_body(a_ref, p0_ref, p1_ref, out_ref, out16_ref):
    v = a_ref[...] + p0_ref[...] + p1_ref[...]
    out_ref[...] = v
    out16_ref[...] = _pack_bf16_pair(v)


def _tc_combine(atom, p0, p1):
    nb = pl.BlockSpec((BN, D), lambda i: (i, 0))
    return pl.pallas_call(
        _combine_body,
        grid=(N // BN,),
        in_specs=[nb, nb, nb],
        out_specs=(nb, pl.BlockSpec((BN, D // 2), lambda i: (i, 0))),
        out_shape=(jax.ShapeDtypeStruct((N, D), jnp.float32),
                   jax.ShapeDtypeStruct((N, D // 2), jnp.int32)),
    )(atom, p0, p1)


def _final_body(a_ref, p0_ref, p1_ref, typ_ref, ng_ref, outW_ref, outb_ref,
                finW_ref, finb_ref, mtab_ref, out_ref):
    x = a_ref[...] + p0_ref[...] + p1_ref[...]
    h = jnp.maximum(
        jnp.dot(x, outW_ref[...], preferred_element_type=jnp.float32)
        + outb_ref[...], 0.0)
    e = jnp.dot(h, finW_ref[...], preferred_element_type=jnp.float32) + finb_ref[...]
    t = typ_ref[...]
    ohm = (t == lax.broadcasted_iota(jnp.int32, (BN, A), 1)).astype(jnp.float32)
    e = e + jnp.dot(ohm, mtab_ref[...], preferred_element_type=jnp.float32)
    g_row = ng_ref[...].reshape(1, BN)
    ohg = (jnp.broadcast_to(g_row, (G, BN))
           == lax.broadcasted_iota(jnp.int32, (G, BN), 0)).astype(jnp.float32)
    partial = jnp.dot(ohg, e, preferred_element_type=jnp.float32)

    @pl.when(pl.program_id(0) == 0)
    def _():
        out_ref[...] = jnp.zeros_like(out_ref)

    out_ref[...] += partial


def _tc_final(atom, p0, p1, atom_types_2d, ng_row, out_W, out_b2, final_W,
              final_b2, mtab):
    nb = pl.BlockSpec((BN, D), lambda i: (i, 0))
    return pl.pallas_call(
        _final_body,
        grid=(N // BN,),
        in_specs=[nb, nb, nb,
                  pl.BlockSpec((BN, 1), lambda i: (i, 0)),
                  pl.BlockSpec((1, 1, BN), lambda i: (i, 0, 0)),
                  _full_spec((D, H)), _full_spec((1, H)), _full_spec((H, 1)),
                  _full_spec((1, 1)), _full_spec((A, 1))],
        out_specs=pl.BlockSpec((G, 1), lambda i: (0, 0)),
        out_shape=jax.ShapeDtypeStruct((G, 1), jnp.float32),
    )(atom, p0, p1, atom_types_2d, ng_row, out_W, out_b2, final_W, final_b2,
      mtab)


def kernel(atom_emb, atom_mean_tab, bond_emb, msg_params, out_W, out_b,
           final_W, final_b, atom_types, bond_types, node_graph_indices,
           connectivity):
    inv = 1.0 / jnp.sqrt(1.0 + EPS)
    dst_idx = connectivity[:, 0].astype(jnp.int32)
    src_idx = connectivity[:, 1].astype(jnp.int32)
    bnd_idx = bond_types.astype(jnp.int32)
    atom_types_2d = atom_types.astype(jnp.int32).reshape(N, 1)
    ng_row = node_graph_indices.astype(jnp.int32).reshape(N // BN, 1, BN)
    kp = jnp.concatenate([jnp.arange(0, BE, 2), jnp.arange(1, BE, 2)])
    k_perm = (jnp.arange(E // BE)[:, None] * BE + kp[None, :]).reshape(E)
    dst_idx_k = dst_idx[k_perm]

    step_weights = []
    for p in msg_params:
        sa = p['atom_bn_gamma'] * inv
        ba = p['atom_bn_beta']
        sb = p['bond_bn_gamma'] * inv
        bb = p['bond_bn_beta']
        W1 = p['W1']
        W1s = sa[:, None] * W1[:D]
        W1t = sa[:, None] * W1[D:2 * D]
        W1b = sb[:, None] * W1[2 * D:]
        hb = (ba @ (W1[:D] + W1[D:2 * D]) + bb @ W1[2 * D:]).reshape(1, 2 * D)
        Wa = sa[:, None] * p['Wa']
        ab = (ba @ p['Wa']).reshape(1, D)
        b2 = p['b2'].reshape(1, D)
        bf = jnp.bfloat16
        W1cat = jnp.concatenate([W1s, W1t, W1b], axis=0).astype(bf)
        step_weights.append((W1cat, p['W2'].astype(bf), Wa.astype(bf),
                             hb, ab, b2))

    atom, atom16 = _tc_init(atom_types_2d, atom_emb)
    bond = None
    p0 = p1 = None
    btyp_k = bnd_idx[k_perm].reshape(E // BE, 1, BE)
    znp = jnp.zeros((NP, D), jnp.float32)
    srcH = (src_idx[:EH], src_idx[EH:])
    dstH = (dst_idx[:EH], dst_idx[EH:])
    dstkH = (dst_idx_k[:EH], dst_idx_k[EH:])
    btypH = (btyp_k[:EH // BE], btyp_k[EH // BE:])
    bondH = [None, None]
    for t in range(len(msg_params)):
        inits = (znp, znp)
        for h in range(2):
            src_rows, dst_rows = _sc_gather(atom16, srcH[h], dstH[h])
            bond_in = (btypH[h], bond_emb) if t == 0 else (bondH[h],)
            msg, bondH[h] = _tc_edge(t == 0, src_rows.reshape(EH // 2, D),
                                     dst_rows.reshape(EH // 2, D), bond_in,
                                     step_weights[t])
            inits = _sc_scatter(msg, dstkH[h], *inits)
        p0, p1 = inits
        if t < len(msg_params) - 1:
            atom, atom16 = _tc_combine(atom, p0, p1)

    return _tc_final(atom, p0, p1, atom_types_2d, ng_row, out_W,
                     out_b.reshape(1, H), final_W, final_b.reshape(1, 1),
                     atom_mean_tab)
